# baseline (speedup 1.0000x reference)
import functools

import jax
import jax.numpy as jnp
from jax import lax
from jax.experimental import pallas as pl
from jax.experimental.pallas import tpu as pltpu
from jax.experimental.pallas import tpu_sc as plsc

_LANES = 16


def _build_sc_call(B, E, N, O):
    info = plsc.get_sparse_core_info()
    NC, NS = info.num_cores, info.num_subcores
    NW = NC * NS
    assert B % NW == 0
    BPW = B // NW
    SPV = _LANES // O
    GPW = BPW // SPV

    mesh = plsc.VectorSubcoreMesh(core_axis_name="c", subcore_axis_name="s")

    @functools.partial(
        pl.kernel,
        mesh=mesh,
        compiler_params=pltpu.CompilerParams(
            use_tc_tiling_on_sc=True,
            needs_layout_passes=False,
            skip_device_barrier=True,
        ),
        out_type=jax.ShapeDtypeStruct((B, O), jnp.float32),
        scratch_types=[
            pltpu.VMEM((E, O, N), jnp.float32),
            pltpu.VMEM((E, O), jnp.float32),
            pltpu.VMEM((BPW,), jnp.int32),
            pltpu.VMEM((BPW,), jnp.int32),
            pltpu.VMEM((BPW, O), jnp.float32),
            pltpu.SemaphoreType.DMA,
            pltpu.SemaphoreType.DMA,
        ],
    )
    def sc_call(w_hbm, b_hbm, idx_hbm, typ_hbm, out_hbm,
                wtab_v, btab_v, idx_v, typ_v, out_v, sem_w, sem_b):
        wid = lax.axis_index("s") * NC + lax.axis_index("c")
        base = wid * BPW
        cw = pltpu.async_copy(w_hbm, wtab_v, sem_w)
        cb = pltpu.async_copy(b_hbm, btab_v, sem_b)
        pltpu.sync_copy(idx_hbm.at[pl.ds(base, BPW)], idx_v)
        pltpu.sync_copy(typ_hbm.at[pl.ds(base, BPW)], typ_v)

        lane = lax.broadcasted_iota(jnp.int32, (_LANES,), 0)
        l4 = lane // SPV
        olane = lane - l4 * SPV

        cb.wait()
        cw.wait()

        @plsc.parallel_loop(0, GPW, step=1, unroll=8)
        def group(r):
            sidx = l4 + r * SPV
            t2 = plsc.load_gather(typ_v, [sidx])
            i2 = plsc.load_gather(idx_v, [sidx])
            wv = plsc.load_gather(wtab_v, [t2, olane, i2])
            bv = plsc.load_gather(btab_v, [t2, olane])
            x = wv + bv
            e = jnp.exp(x * 2.0)
            y = 1.0 - 2.0 / (e + 1.0)
            plsc.store_scatter(out_v, [sidx, olane], y)

        pltpu.sync_copy(out_v, out_hbm.at[pl.ds(base, BPW)])

    return sc_call


def kernel(W, b, action_indecies, action_n_obj, action_types):
    del action_n_obj
    E, O, N = W.shape
    B = action_indecies.shape[0]
    idx = action_indecies.astype(jnp.int32)
    typ = action_types.astype(jnp.int32)
    return _build_sc_call(B, E, N, O)(W, b, idx, typ)


# speedup vs baseline: 1.0371x; 1.0371x over previous
import functools

import jax
import jax.numpy as jnp
from jax import lax
from jax.experimental import pallas as pl
from jax.experimental.pallas import tpu as pltpu
from jax.experimental.pallas import tpu_sc as plsc

_LANES = 16


def _build_sc_call(B, E, N, O):
    info = plsc.get_sparse_core_info()
    NC, NS = info.num_cores, info.num_subcores
    NW = NC * NS
    assert B % NW == 0
    BPW = B // NW
    SPV = _LANES // O
    GPW = BPW // SPV

    mesh = plsc.VectorSubcoreMesh(core_axis_name="c", subcore_axis_name="s")

    @functools.partial(
        pl.kernel,
        mesh=mesh,
        compiler_params=pltpu.CompilerParams(
            use_tc_tiling_on_sc=False,
            needs_layout_passes=False,
            skip_device_barrier=True,
        ),
        out_type=jax.ShapeDtypeStruct((B, O), jnp.float32),
        scratch_types=[
            pltpu.VMEM((E, O, N), jnp.float32),
            pltpu.VMEM((E, O), jnp.float32),
            pltpu.VMEM((BPW,), jnp.int32),
            pltpu.VMEM((BPW,), jnp.int32),
            pltpu.VMEM((BPW, O), jnp.float32),
            pltpu.SemaphoreType.DMA,
            pltpu.SemaphoreType.DMA,
        ],
    )
    def sc_call(w_hbm, b_hbm, idx_hbm, typ_hbm, out_hbm,
                wtab_v, btab_v, idx_v, typ_v, out_v, sem_w, sem_b):
        wid = lax.axis_index("s") * NC + lax.axis_index("c")
        base = wid * BPW
        cw = pltpu.async_copy(w_hbm, wtab_v, sem_w)
        cb = pltpu.async_copy(b_hbm, btab_v, sem_b)
        pltpu.sync_copy(idx_hbm.at[pl.ds(base, BPW)], idx_v)
        pltpu.sync_copy(typ_hbm.at[pl.ds(base, BPW)], typ_v)

        lane = lax.broadcasted_iota(jnp.int32, (_LANES,), 0)
        l4 = lane // SPV
        olane = lane - l4 * SPV

        cb.wait()
        cw.wait()

        @plsc.parallel_loop(0, GPW, step=1, unroll=4)
        def group(r):
            sidx = l4 + r * SPV
            t2 = plsc.load_gather(typ_v, [sidx])
            i2 = plsc.load_gather(idx_v, [sidx])
            wv = plsc.load_gather(wtab_v, [t2, olane, i2])
            bv = plsc.load_gather(btab_v, [t2, olane])
            x = wv + bv
            e = jnp.exp(x * 2.0)
            y = 1.0 - 2.0 / (e + 1.0)
            plsc.store_scatter(out_v, [sidx, olane], y)

        pltpu.sync_copy(out_v, out_hbm.at[pl.ds(base, BPW)])

    return sc_call


def kernel(W, b, action_indecies, action_n_obj, action_types):
    del action_n_obj
    E, O, N = W.shape
    B = action_indecies.shape[0]
    idx = action_indecies.astype(jnp.int32)
    typ = action_types.astype(jnp.int32)
    return _build_sc_call(B, E, N, O)(W, b, idx, typ)


# speedup vs baseline: 1.1620x; 1.1205x over previous
import functools

import jax
import jax.numpy as jnp
from jax import lax
from jax.experimental import pallas as pl
from jax.experimental.pallas import tpu as pltpu
from jax.experimental.pallas import tpu_sc as plsc

_LANES = 16


def _build_sc_call(B, E, N, O):
    info = plsc.get_sparse_core_info()
    NC, NS = info.num_cores, info.num_subcores
    NW = NC * NS
    assert B % NW == 0
    BPW = B // NW
    SPV = _LANES // O
    GPW = BPW // SPV

    mesh = plsc.VectorSubcoreMesh(core_axis_name="c", subcore_axis_name="s")

    @functools.partial(
        pl.kernel,
        mesh=mesh,
        compiler_params=pltpu.CompilerParams(
            use_tc_tiling_on_sc=False,
            needs_layout_passes=False,
            skip_device_barrier=True,
        ),
        out_type=jax.ShapeDtypeStruct((NW, O, BPW), jnp.float32),
        scratch_types=[
            pltpu.VMEM((E, N // 128, O, 128), jnp.float32),
            pltpu.VMEM((E, O), jnp.float32),
            pltpu.VMEM((BPW,), jnp.int32),
            pltpu.VMEM((BPW,), jnp.int32),
            pltpu.VMEM((O, BPW), jnp.float32),
            pltpu.SemaphoreType.DMA,
            pltpu.SemaphoreType.DMA,
        ],
    )
    def sc_call(w_hbm, b_hbm, idx_hbm, typ_hbm, out_hbm,
                wtab_v, btab_v, idx_v, typ_v, out_v, sem_w, sem_b):
        wid = lax.axis_index("s") * NC + lax.axis_index("c")
        base = wid * BPW
        cw = pltpu.async_copy(w_hbm, wtab_v, sem_w)
        cb = pltpu.async_copy(b_hbm, btab_v, sem_b)
        pltpu.sync_copy(idx_hbm.at[pl.ds(base, BPW)], idx_v)
        pltpu.sync_copy(typ_hbm.at[pl.ds(base, BPW)], typ_v)

        lane = lax.broadcasted_iota(jnp.int32, (_LANES,), 0)
        l4 = lane // SPV
        olane = lane - l4 * SPV

        cb.wait()
        cw.wait()

        @plsc.parallel_loop(0, GPW, step=1, unroll=4)
        def group(r):
            sidx = l4 + r * SPV
            t2 = plsc.load_gather(typ_v, [sidx])
            i2 = plsc.load_gather(idx_v, [sidx])
            wv = plsc.load_gather(
                wtab_v,
                [t2, lax.shift_right_logical(i2, 7), olane, i2 & 127],
            )
            bv = plsc.load_gather(btab_v, [t2, olane])
            x = wv + bv
            e = jnp.exp(x * 2.0)
            y = 1.0 - 2.0 / (e + 1.0)
            plsc.store_scatter(out_v, [olane, sidx], y)

        pltpu.sync_copy(out_v, out_hbm.at[wid])

    return sc_call


def kernel(W, b, action_indecies, action_n_obj, action_types):
    del action_n_obj
    E, O, N = W.shape
    B = action_indecies.shape[0]
    idx = action_indecies.astype(jnp.int32)
    typ = action_types.astype(jnp.int32)
    w4 = W.reshape(E, O, N // 128, 128).transpose(0, 2, 1, 3)
    out = _build_sc_call(B, E, N, O)(w4, b, idx, typ)
    return out.transpose(0, 2, 1).reshape(B, O)


# speedup vs baseline: 1.1623x; 1.0003x over previous
import functools

import jax
import jax.numpy as jnp
from jax import lax
from jax.experimental import pallas as pl
from jax.experimental.pallas import tpu as pltpu
from jax.experimental.pallas import tpu_sc as plsc

_LANES = 16


def _build_sc_call(B, E, N, O):
    info = plsc.get_sparse_core_info()
    NC, NS = info.num_cores, info.num_subcores
    NW = NC * NS
    assert B % NW == 0
    BPW = B // NW
    SPV = _LANES // O
    GPW = BPW // SPV

    mesh = plsc.VectorSubcoreMesh(core_axis_name="c", subcore_axis_name="s")

    @functools.partial(
        pl.kernel,
        mesh=mesh,
        compiler_params=pltpu.CompilerParams(
            use_tc_tiling_on_sc=False,
            needs_layout_passes=False,
            skip_device_barrier=True,
        ),
        out_type=jax.ShapeDtypeStruct((NW, O, BPW), jnp.float32),
        scratch_types=[
            pltpu.VMEM((E, N // 128, O, 128), jnp.float32),
            pltpu.VMEM((O, E), jnp.float32),
            pltpu.VMEM((BPW,), jnp.int32),
            pltpu.VMEM((BPW,), jnp.int32),
            pltpu.VMEM((O, BPW), jnp.float32),
            pltpu.SemaphoreType.DMA,
            pltpu.SemaphoreType.DMA,
        ],
    )
    def sc_call(w_hbm, b_hbm, idx_hbm, typ_hbm, out_hbm,
                wtab_v, btab_v, idx_v, typ_v, out_v, sem_w, sem_b):
        wid = lax.axis_index("s") * NC + lax.axis_index("c")
        base = wid * BPW
        cw = pltpu.async_copy(w_hbm, wtab_v, sem_w)
        cb = pltpu.async_copy(b_hbm, btab_v, sem_b)
        pltpu.sync_copy(idx_hbm.at[pl.ds(base, BPW)], idx_v)
        pltpu.sync_copy(typ_hbm.at[pl.ds(base, BPW)], typ_v)

        lane = lax.broadcasted_iota(jnp.int32, (_LANES,), 0)
        l4 = lane // SPV
        olane = lane - l4 * SPV

        cb.wait()
        cw.wait()

        @plsc.parallel_loop(0, GPW, step=1, unroll=4)
        def group(r):
            sidx = l4 + r * SPV
            t2 = plsc.load_gather(typ_v, [sidx])
            i2 = plsc.load_gather(idx_v, [sidx])
            wv = plsc.load_gather(
                wtab_v,
                [t2, lax.shift_right_logical(i2, 7), olane, i2 & 127],
            )
            bv = plsc.load_gather(btab_v, [olane, t2])
            x = wv + bv
            e = jnp.exp(x * 2.0)
            y = 1.0 - 2.0 / (e + 1.0)
            plsc.store_scatter(out_v, [olane, sidx], y)

        pltpu.sync_copy(out_v, out_hbm.at[wid])

    return sc_call


def kernel(W, b, action_indecies, action_n_obj, action_types):
    del action_n_obj
    E, O, N = W.shape
    B = action_indecies.shape[0]
    idx = action_indecies.astype(jnp.int32)
    typ = action_types.astype(jnp.int32)
    w4 = W.reshape(E, O, N // 128, 128).transpose(0, 2, 1, 3)
    out = _build_sc_call(B, E, N, O)(w4, b.T, idx, typ)
    return out.transpose(0, 2, 1).reshape(B, O)


# speedup vs baseline: 1.1707x; 1.0072x over previous
import functools

import jax
import jax.numpy as jnp
from jax import lax
from jax.experimental import pallas as pl
from jax.experimental.pallas import tpu as pltpu
from jax.experimental.pallas import tpu_sc as plsc

_LANES = 16


def _build_sc_call(B, E, N, O):
    info = plsc.get_sparse_core_info()
    NC, NS = info.num_cores, info.num_subcores
    NW = NC * NS
    assert B % NW == 0
    BPW = B // NW
    SPV = _LANES // O
    GPW = BPW // SPV

    mesh = plsc.VectorSubcoreMesh(core_axis_name="c", subcore_axis_name="s")

    @functools.partial(
        pl.kernel,
        mesh=mesh,
        compiler_params=pltpu.CompilerParams(
            use_tc_tiling_on_sc=False,
            needs_layout_passes=False,
            skip_device_barrier=True,
        ),
        out_type=jax.ShapeDtypeStruct((NW, O, BPW), jnp.float32),
        scratch_types=[
            pltpu.VMEM((E, N // 128, O, 128), jnp.float32),
            pltpu.VMEM((O, E), jnp.float32),
            pltpu.VMEM((BPW,), jnp.int32),
            pltpu.VMEM((BPW,), jnp.int32),
            pltpu.VMEM((O, BPW), jnp.float32),
            pltpu.SemaphoreType.DMA,
            pltpu.SemaphoreType.DMA,
        ],
    )
    def sc_call(w_hbm, b_hbm, idx_hbm, typ_hbm, out_hbm,
                wtab_v, btab_v, idx_v, typ_v, out_v, sem_w, sem_b):
        wid = lax.axis_index("s") * NC + lax.axis_index("c")
        base = wid * BPW
        cw = pltpu.async_copy(w_hbm, wtab_v, sem_w)
        cb = pltpu.async_copy(b_hbm, btab_v, sem_b)
        pltpu.sync_copy(idx_hbm.at[pl.ds(base, BPW)], idx_v)
        pltpu.sync_copy(typ_hbm.at[pl.ds(base, BPW)], typ_v)

        lane = lax.broadcasted_iota(jnp.int32, (_LANES,), 0)
        l4 = lane // SPV
        olane = lane - l4 * SPV

        cb.wait()
        cw.wait()

        @plsc.parallel_loop(0, GPW, step=1, unroll=2)
        def group(r):
            sidx = l4 + r * SPV
            t2 = plsc.load_gather(typ_v, [sidx])
            i2 = plsc.load_gather(idx_v, [sidx])
            wv = plsc.load_gather(
                wtab_v,
                [t2, lax.shift_right_logical(i2, 7), olane, i2 & 127],
            )
            bv = plsc.load_gather(btab_v, [olane, t2])
            x = wv + bv
            e = jnp.exp(x * 2.0)
            y = 1.0 - 2.0 / (e + 1.0)
            plsc.store_scatter(out_v, [olane, sidx], y)

        pltpu.sync_copy(out_v, out_hbm.at[wid])

    return sc_call


def kernel(W, b, action_indecies, action_n_obj, action_types):
    del action_n_obj
    E, O, N = W.shape
    B = action_indecies.shape[0]
    idx = action_indecies.astype(jnp.int32)
    typ = action_types.astype(jnp.int32)
    w4 = W.reshape(E, O, N // 128, 128).transpose(0, 2, 1, 3)
    out = _build_sc_call(B, E, N, O)(w4, b.T, idx, typ)
    return out.transpose(0, 2, 1).reshape(B, O)
